# baseline (speedup 1.0000x reference)
import functools

import jax
import jax.numpy as jnp
from jax import lax
from jax.experimental import pallas as pl
from jax.experimental.pallas import tpu as pltpu
from jax.experimental.pallas import tpu_sc as plsc

B = 16384
D = 16
NC, NS = 2, 16
NW = NC * NS
BPW = B // NW
CHUNK = 128
NCHUNK = BPW // CHUNK
NPASS = 2
CPP = NCHUNK // NPASS
PASS_ROWS = BPW // NPASS


def _srsqrt(x):
    i = lax.bitcast_convert_type(x, jnp.int32)
    y = lax.bitcast_convert_type(jnp.int32(0x5F3759DF) - (i >> 1),
                                 jnp.float32)
    for _ in range(3):
        y = y * (jnp.float32(1.5) - jnp.float32(0.5) * x * y * y)
    return y


def _hsum(x):
    for w in (8, 4, 2):
        x = x[:w] + x[w:2 * w]
    return x[0] + x[1]


def _fb_body(uidx_hbm, midx_hbm, uemb_hbm, memb_hbm, ubias_hbm, mbias_hbm,
             out_hbm, uidxr, midxr, uidxh, midxh, uidxf, midxf,
             ubuf, mbuf, ub_v, mb_v, out_v, sem_e, sem_b):
    wid = lax.axis_index("s") * NC + lax.axis_index("c")
    base = wid * BPW

    for c in range(NCHUNK):
        pltpu.sync_copy(uidx_hbm.at[pl.ds(base + c * CHUNK, CHUNK)],
                        uidxr.at[c])
        pltpu.sync_copy(midx_hbm.at[pl.ds(base + c * CHUNK, CHUNK)],
                        midxr.at[c])
    pltpu.sync_copy(uidx_hbm.at[pl.ds(base, BPW)], uidxf)
    pltpu.sync_copy(midx_hbm.at[pl.ds(base, BPW)], midxf)

    for c in range(NCHUNK):
        for k in range(CHUNK // D):
            sl = pl.ds(k * D, D)
            uidxh[c, sl] = uidxr[c, sl] >> 3
            midxh[c, sl] = midxr[c, sl] >> 3

    bias_copies = []
    for c in range(NCHUNK):
        sl = pl.ds(c * CHUNK, CHUNK)
        bias_copies.append(
            pltpu.async_copy(ubias_hbm.at[uidxr.at[c]], ub_v.at[sl], sem_b))
        bias_copies.append(
            pltpu.async_copy(mbias_hbm.at[midxr.at[c]], mb_v.at[sl], sem_b))

    lane = lax.iota(jnp.int32, 16)

    def fire(p):
        copies = []
        for h in range(CPP):
            c = p * CPP + h
            sl = pl.ds(h * CHUNK, CHUNK)
            copies.append(
                pltpu.async_copy(uemb_hbm.at[uidxh.at[c]], ubuf.at[sl],
                                 sem_e))
            copies.append(
                pltpu.async_copy(memb_hbm.at[midxh.at[c]], mbuf.at[sl],
                                 sem_e))
        return copies

    emb_copies = fire(0)
    for cp in bias_copies:
        cp.wait()

    for p in range(NPASS):
        for cp in emb_copies:
            cp.wait()

        def tile(t, carry, p=p):
            g0 = p * PASS_ROWS + t * D
            raw_u = uidxf[pl.ds(g0, D)]
            raw_m = midxf[pl.ds(g0, D)]
            cb_u = (raw_u & 7) * D
            cb_m = (raw_m & 7) * D
            sim = jnp.zeros((16,), jnp.float32)
            for r in range(D):
                urow = ubuf[t * D + r, pl.ds(cb_u[r], D)]
                mrow = mbuf[t * D + r, pl.ds(cb_m[r], D)]
                du = _hsum(urow * mrow)
                us = jnp.maximum(_hsum(urow * urow), jnp.float32(1e-30))
                ms = jnp.maximum(_hsum(mrow * mrow), jnp.float32(1e-30))
                rs_u = _srsqrt(us)
                rs_m = _srsqrt(ms)
                ru = jnp.where(us * rs_u >= jnp.float32(1e-8), rs_u,
                               jnp.float32(1e8))
                rm = jnp.where(ms * rs_m >= jnp.float32(1e-8), rs_m,
                               jnp.float32(1e8))
                simr = du * ru * rm * jnp.float32(2.5) + jnp.float32(2.75)
                sim = jnp.where(lane == r, simr, sim)
            sl = pl.ds(g0, D)
            out_v[sl] = sim + ub_v[sl] + mb_v[sl]
            return carry

        lax.fori_loop(0, PASS_ROWS // D, tile, 0)
        if p + 1 < NPASS:
            emb_copies = fire(p + 1)

    pltpu.sync_copy(out_v, out_hbm.at[pl.ds(base, BPW)])


_fb_kernel = functools.partial(
    pl.kernel,
    out_type=jax.ShapeDtypeStruct((B,), jnp.float32),
    mesh=plsc.VectorSubcoreMesh(core_axis_name="c", subcore_axis_name="s"),
    compiler_params=pltpu.CompilerParams(needs_layout_passes=True),
    scratch_types=[
        pltpu.VMEM((NCHUNK, CHUNK), jnp.int32),
        pltpu.VMEM((NCHUNK, CHUNK), jnp.int32),
        pltpu.VMEM((NCHUNK, CHUNK), jnp.int32),
        pltpu.VMEM((NCHUNK, CHUNK), jnp.int32),
        pltpu.VMEM((BPW,), jnp.int32),
        pltpu.VMEM((BPW,), jnp.int32),
        pltpu.VMEM((PASS_ROWS, 128), jnp.float32),
        pltpu.VMEM((PASS_ROWS, 128), jnp.float32),
        pltpu.VMEM((BPW,), jnp.float32),
        pltpu.VMEM((BPW,), jnp.float32),
        pltpu.VMEM((BPW,), jnp.float32),
        pltpu.SemaphoreType.DMA,
        pltpu.SemaphoreType.DMA,
    ],
)(_fb_body)


def kernel(user_idx, movie_idx, user_embeds, movie_embeds, user_biases,
           movie_biases):
    c = jnp.float32(1.0000001)
    return _fb_kernel(user_idx.astype(jnp.int32),
                      movie_idx.astype(jnp.int32),
                      (user_embeds * c).reshape(-1, 128),
                      (movie_embeds * c).reshape(-1, 128),
                      jnp.squeeze(user_biases, -1),
                      jnp.squeeze(movie_biases, -1))


# speedup vs baseline: 1.0929x; 1.0929x over previous
import functools

import jax
import jax.numpy as jnp
from jax import lax
from jax.experimental import pallas as pl
from jax.experimental.pallas import tpu as pltpu
from jax.experimental.pallas import tpu_sc as plsc

B = 16384
D = 16
NC, NS = 2, 16
NW = NC * NS
BPW = B // NW
CHUNK = 128
NCHUNK = BPW // CHUNK


def _nrsqrt(x):
    i = plsc.bitcast(x, jnp.int32)
    y = plsc.bitcast(jnp.int32(0x5F3759DF) - (i >> 1), jnp.float32)
    for _ in range(3):
        y = y * (1.5 - 0.5 * x * y * y)
    return y


def _fb_body(uidx_hbm, midx_hbm, uemb_hbm, memb_hbm, ubias_hbm, mbias_hbm,
             out_hbm, uidx_v, midx_v, urows_v, mrows_v, ub_v, mb_v, out_v,
             sem):
    wid = lax.axis_index("s") * NC + lax.axis_index("c")
    base = wid * BPW

    for c in range(NCHUNK):
        pltpu.sync_copy(uidx_hbm.at[pl.ds(base + c * CHUNK, CHUNK)],
                        uidx_v.at[c])
        pltpu.sync_copy(midx_hbm.at[pl.ds(base + c * CHUNK, CHUNK)],
                        midx_v.at[c])

    copies = []
    for c in range(NCHUNK):
        sl = pl.ds(c * CHUNK, CHUNK)
        copies.append(pltpu.async_copy(uemb_hbm.at[uidx_v.at[c]],
                                       urows_v.at[sl], sem))
        copies.append(pltpu.async_copy(memb_hbm.at[midx_v.at[c]],
                                       mrows_v.at[sl], sem))
        copies.append(pltpu.async_copy(ubias_hbm.at[uidx_v.at[c]],
                                       ub_v.at[sl], sem))
        copies.append(pltpu.async_copy(mbias_hbm.at[midx_v.at[c]],
                                       mb_v.at[sl], sem))
    for cp in copies:
        cp.wait()

    lane = lax.iota(jnp.int32, 16)

    def tile(t, carry):
        rows = t * 16 + lane
        dot = jnp.zeros((16,), jnp.float32)
        uu = jnp.zeros((16,), jnp.float32)
        mm = jnp.zeros((16,), jnp.float32)
        for j in range(D):
            cols = jnp.full((16,), j, jnp.int32)
            uc = plsc.load_gather(urows_v, [rows, cols])
            mc = plsc.load_gather(mrows_v, [rows, cols])
            dot = dot + uc * mc
            uu = uu + uc * uc
            mm = mm + mc * mc
        uu = jnp.maximum(uu, 1e-30)
        mm = jnp.maximum(mm, 1e-30)
        nu = jnp.maximum(uu * _nrsqrt(uu), 1e-8)
        nm = jnp.maximum(mm * _nrsqrt(mm), 1e-8)
        sim = dot / (nu * nm) * 2.5 + 2.75
        sl = pl.ds(t * 16, 16)
        out_v[sl] = sim + ub_v[sl] + mb_v[sl]
        return carry

    lax.fori_loop(0, BPW // 16, tile, 0)

    pltpu.sync_copy(out_v, out_hbm.at[pl.ds(base, BPW)])


_fb_kernel = functools.partial(
    pl.kernel,
    out_type=jax.ShapeDtypeStruct((B,), jnp.float32),
    mesh=plsc.VectorSubcoreMesh(core_axis_name="c", subcore_axis_name="s"),
    compiler_params=pltpu.CompilerParams(needs_layout_passes=False,
                                         use_tc_tiling_on_sc=False),
    scratch_types=[
        pltpu.VMEM((NCHUNK, CHUNK), jnp.int32),
        pltpu.VMEM((NCHUNK, CHUNK), jnp.int32),
        pltpu.VMEM((BPW, D), jnp.float32),
        pltpu.VMEM((BPW, D), jnp.float32),
        pltpu.VMEM((BPW,), jnp.float32),
        pltpu.VMEM((BPW,), jnp.float32),
        pltpu.VMEM((BPW,), jnp.float32),
        pltpu.SemaphoreType.DMA,
    ],
)(_fb_body)


def kernel(user_idx, movie_idx, user_embeds, movie_embeds, user_biases,
           movie_biases):
    return _fb_kernel(user_idx.astype(jnp.int32),
                      movie_idx.astype(jnp.int32),
                      user_embeds, movie_embeds,
                      jnp.squeeze(user_biases, -1),
                      jnp.squeeze(movie_biases, -1))
